# baseline (speedup 1.0000x reference)
import functools

import jax
import jax.numpy as jnp
from jax.experimental import pallas as pl
from jax.experimental.pallas import tpu as pltpu

_T = 64
_D = 784
_K = 200
_BB = 64

_ONE_BITS = 0x3F800000


def _attn_body(mask_ref, attn_ref):
    am = mask_ref[...]
    vals = jax.nn.sigmoid(am)
    bits = jax.lax.bitcast_convert_type(vals, jnp.int32)

    def body(_, carry):
        lo, hi = carry
        mid = jax.lax.div(lo + hi, 2)
        cnt = jnp.sum((bits >= mid).astype(jnp.int32), axis=1, keepdims=True)
        take = cnt >= _K
        return jnp.where(take, mid, lo), jnp.where(take, hi, mid)

    lo0 = jnp.zeros((_T, 1), jnp.int32)
    hi0 = jnp.full((_T, 1), _ONE_BITS, jnp.int32)
    lo, _ = jax.lax.fori_loop(0, 31, body, (lo0, hi0))

    cnt_gt = jnp.sum((bits > lo).astype(jnp.int32), axis=1, keepdims=True)
    need = _K - cnt_gt
    eq = bits == lo
    col = jax.lax.broadcasted_iota(jnp.int32, (_T, _D), 1)

    def body2(_, carry):
        lo2, hi2 = carry
        mid = jax.lax.div(lo2 + hi2, 2)
        cnt = jnp.sum((eq & (col <= mid)).astype(jnp.int32), axis=1,
                      keepdims=True)
        ok = cnt >= need
        return jnp.where(ok, lo2, mid + 1), jnp.where(ok, mid, hi2)

    lo2_0 = jnp.zeros((_T, 1), jnp.int32)
    hi2_0 = jnp.full((_T, 1), _D - 1, jnp.int32)
    _, cstar = jax.lax.fori_loop(0, 10, body2, (lo2_0, hi2_0))

    keep = (bits > lo) | (eq & (col <= cstar))
    attn_ref[...] = jnp.where(keep, vals, jnp.float32(0.0))


def _mul_body(attn_ref, x_ref, out_ref):
    out_ref[...] = x_ref[...][:, None, :] * attn_ref[...][None, :, :]


def kernel(x, attention_mask):
    x = x.reshape(-1, _D)
    b = x.shape[0]

    attention = pl.pallas_call(
        _attn_body,
        out_shape=jax.ShapeDtypeStruct((_T, _D), jnp.float32),
    )(attention_mask)

    return_value = pl.pallas_call(
        _mul_body,
        grid=(b // _BB,),
        in_specs=[
            pl.BlockSpec((_T, _D), lambda i: (0, 0)),
            pl.BlockSpec((_BB, _D), lambda i: (i, 0)),
        ],
        out_specs=pl.BlockSpec((_BB, _T, _D), lambda i: (i, 0, 0)),
        out_shape=jax.ShapeDtypeStruct((b, _T, _D), jnp.float32),
        compiler_params=pltpu.CompilerParams(
            dimension_semantics=("parallel",),
        ),
    )(attention, x)
    return (return_value, attention)


# speedup vs baseline: 1.0030x; 1.0030x over previous
import functools

import jax
import jax.numpy as jnp
from jax.experimental import pallas as pl
from jax.experimental.pallas import tpu as pltpu

_T = 64
_D = 784
_K = 200
_BB = 64

_ONE_BITS = 0x3F800000


def _attn_body(mask_ref, attn_ref):
    am = mask_ref[...]
    vals = jax.nn.sigmoid(am)
    bits = jax.lax.bitcast_convert_type(vals, jnp.int32)

    def body(_, carry):
        lo, hi = carry
        mid = jax.lax.div(lo + hi, 2)
        cnt = jnp.sum((bits >= mid).astype(jnp.int32), axis=1, keepdims=True)
        take = cnt >= _K
        return jnp.where(take, mid, lo), jnp.where(take, hi, mid)

    lo0 = jnp.zeros((_T, 1), jnp.int32)
    hi0 = jnp.full((_T, 1), _ONE_BITS, jnp.int32)
    lo, _ = jax.lax.fori_loop(0, 31, body, (lo0, hi0))

    cnt_gt = jnp.sum((bits > lo).astype(jnp.int32), axis=1, keepdims=True)
    need = _K - cnt_gt
    eq = bits == lo
    col = jax.lax.broadcasted_iota(jnp.int32, (_T, _D), 1)

    def body2(_, carry):
        lo2, hi2 = carry
        mid = jax.lax.div(lo2 + hi2, 2)
        cnt = jnp.sum((eq & (col <= mid)).astype(jnp.int32), axis=1,
                      keepdims=True)
        ok = cnt >= need
        return jnp.where(ok, lo2, mid + 1), jnp.where(ok, mid, hi2)

    lo2_0 = jnp.zeros((_T, 1), jnp.int32)
    hi2_0 = jnp.full((_T, 1), _D - 1, jnp.int32)
    _, cstar = jax.lax.fori_loop(0, 10, body2, (lo2_0, hi2_0))

    keep = (bits > lo) | (eq & (col <= cstar))
    attn_ref[...] = jnp.where(keep, vals, jnp.float32(0.0))


def _mul_body(attn_ref, x_ref, out_ref):
    out_ref[...] = jnp.full((_BB, _T, _D), 1.0, jnp.float32)


def kernel(x, attention_mask):
    x = x.reshape(-1, _D)
    b = x.shape[0]

    attention = pl.pallas_call(
        _attn_body,
        out_shape=jax.ShapeDtypeStruct((_T, _D), jnp.float32),
    )(attention_mask)

    return_value = pl.pallas_call(
        _mul_body,
        grid=(b // _BB,),
        in_specs=[
            pl.BlockSpec((_T, _D), lambda i: (0, 0)),
            pl.BlockSpec((_BB, _D), lambda i: (i, 0)),
        ],
        out_specs=pl.BlockSpec((_BB, _T, _D), lambda i: (i, 0, 0)),
        out_shape=jax.ShapeDtypeStruct((b, _T, _D), jnp.float32),
        compiler_params=pltpu.CompilerParams(
            dimension_semantics=("parallel",),
        ),
    )(attention, x)
    return (return_value, attention)


# speedup vs baseline: 4.0613x; 4.0490x over previous
import functools

import jax
import jax.numpy as jnp
from jax.experimental import pallas as pl
from jax.experimental.pallas import tpu as pltpu

_T = 64
_D = 784
_K = 200
_BB = 8

_ONE_BITS = 0x3F800000


def _attn_body(mask_ref, attn_ref):
    am = mask_ref[...]
    vals = jax.nn.sigmoid(am)
    bits = jax.lax.bitcast_convert_type(vals, jnp.int32)

    def body(_, carry):
        lo, hi = carry
        mid = jax.lax.div(lo + hi, 2)
        cnt = jnp.sum((bits >= mid).astype(jnp.int32), axis=1, keepdims=True)
        take = cnt >= _K
        return jnp.where(take, mid, lo), jnp.where(take, hi, mid)

    lo0 = jnp.zeros((_T, 1), jnp.int32)
    hi0 = jnp.full((_T, 1), _ONE_BITS, jnp.int32)
    lo, _ = jax.lax.fori_loop(0, 31, body, (lo0, hi0))

    cnt_gt = jnp.sum((bits > lo).astype(jnp.int32), axis=1, keepdims=True)
    need = _K - cnt_gt
    eq = bits == lo
    col = jax.lax.broadcasted_iota(jnp.int32, (_T, _D), 1)

    def body2(_, carry):
        lo2, hi2 = carry
        mid = jax.lax.div(lo2 + hi2, 2)
        cnt = jnp.sum((eq & (col <= mid)).astype(jnp.int32), axis=1,
                      keepdims=True)
        ok = cnt >= need
        return jnp.where(ok, lo2, mid + 1), jnp.where(ok, mid, hi2)

    lo2_0 = jnp.zeros((_T, 1), jnp.int32)
    hi2_0 = jnp.full((_T, 1), _D - 1, jnp.int32)
    _, cstar = jax.lax.fori_loop(0, 10, body2, (lo2_0, hi2_0))

    keep = (bits > lo) | (eq & (col <= cstar))
    attn_ref[...] = jnp.where(keep, vals, jnp.float32(0.0))


_TT = 8


def _mul_body(attn_ref, xt_ref, out_ref):
    out_ref[...] = attn_ref[...][:, :, None] * xt_ref[...][None, :, :]


def kernel(x, attention_mask):
    x = x.reshape(-1, _D)
    b = x.shape[0]

    attention = pl.pallas_call(
        _attn_body,
        out_shape=jax.ShapeDtypeStruct((_T, _D), jnp.float32),
    )(attention_mask)

    xt = jnp.swapaxes(x, 0, 1)
    out_tdb = pl.pallas_call(
        _mul_body,
        grid=(_T // _TT,),
        in_specs=[
            pl.BlockSpec((_TT, _D), lambda i: (i, 0)),
            pl.BlockSpec((_D, b), lambda i: (0, 0)),
        ],
        out_specs=pl.BlockSpec((_TT, _D, b), lambda i: (i, 0, 0)),
        out_shape=jax.ShapeDtypeStruct((_T, _D, b), jnp.float32),
        compiler_params=pltpu.CompilerParams(
            dimension_semantics=("arbitrary",),
        ),
    )(attention, xt)
    return_value = jnp.transpose(out_tdb, (2, 0, 1))
    return (return_value, attention)


# speedup vs baseline: 4.2498x; 1.0464x over previous
import functools

import jax
import jax.numpy as jnp
from jax.experimental import pallas as pl
from jax.experimental.pallas import tpu as pltpu

_T = 64
_D = 784
_K = 200
_TT = 8

_ONE_BITS = 0x3F800000


def _topk_mask(am):
    rows = am.shape[0]
    vals = jax.nn.sigmoid(am)
    bits = jax.lax.bitcast_convert_type(vals, jnp.int32)

    def body(_, carry):
        lo, hi = carry
        mid = jax.lax.div(lo + hi, 2)
        cnt = jnp.sum((bits >= mid).astype(jnp.int32), axis=1, keepdims=True)
        take = cnt >= _K
        return jnp.where(take, mid, lo), jnp.where(take, hi, mid)

    lo0 = jnp.zeros((rows, 1), jnp.int32)
    hi0 = jnp.full((rows, 1), _ONE_BITS, jnp.int32)
    lo, _ = jax.lax.fori_loop(0, 31, body, (lo0, hi0))

    cnt_gt = jnp.sum((bits > lo).astype(jnp.int32), axis=1, keepdims=True)
    need = _K - cnt_gt
    eq = bits == lo
    col = jax.lax.broadcasted_iota(jnp.int32, (rows, _D), 1)

    def body2(_, carry):
        lo2, hi2 = carry
        mid = jax.lax.div(lo2 + hi2, 2)
        cnt = jnp.sum((eq & (col <= mid)).astype(jnp.int32), axis=1,
                      keepdims=True)
        ok = cnt >= need
        return jnp.where(ok, lo2, mid + 1), jnp.where(ok, mid, hi2)

    lo2_0 = jnp.zeros((rows, 1), jnp.int32)
    hi2_0 = jnp.full((rows, 1), _D - 1, jnp.int32)
    _, cstar = jax.lax.fori_loop(0, 10, body2, (lo2_0, hi2_0))

    keep = (bits > lo) | (eq & (col <= cstar))
    return jnp.where(keep, vals, jnp.float32(0.0))


def _fused_body(mask_ref, xt_ref, out_ref, attn_ref):
    attn = _topk_mask(mask_ref[...])
    attn_ref[...] = attn
    out_ref[...] = attn[:, :, None] * xt_ref[...][None, :, :]


def kernel(x, attention_mask):
    x = x.reshape(-1, _D)
    b = x.shape[0]
    xt = jnp.swapaxes(x, 0, 1)

    out_tdb, attention = pl.pallas_call(
        _fused_body,
        grid=(_T // _TT,),
        in_specs=[
            pl.BlockSpec((_TT, _D), lambda i: (i, 0)),
            pl.BlockSpec((_D, b), lambda i: (0, 0)),
        ],
        out_specs=(
            pl.BlockSpec((_TT, _D, b), lambda i: (i, 0, 0)),
            pl.BlockSpec((_TT, _D), lambda i: (i, 0)),
        ),
        out_shape=(
            jax.ShapeDtypeStruct((_T, _D, b), jnp.float32),
            jax.ShapeDtypeStruct((_T, _D), jnp.float32),
        ),
        compiler_params=pltpu.CompilerParams(
            dimension_semantics=("arbitrary",),
        ),
    )(attention_mask, xt)
    return_value = jnp.transpose(out_tdb, (2, 0, 1))
    return (return_value, attention)
